# baseline (speedup 1.0000x reference)
import functools

import jax
import jax.numpy as jnp
from jax import lax
from jax.experimental import pallas as pl
from jax.experimental.pallas import tpu as pltpu
from jax.experimental.pallas import tpu_sc as plsc

_NC = 2
_NS = 16
_L = 16
_CHUNK = 48


def _proj_body(x_ref, wl_ref, wr_ref, bl_ref, br_ref, xl_ref, xr_ref):
    xb = x_ref[...]
    xl_ref[...] = (
        jnp.dot(xb, wl_ref[...], preferred_element_type=jnp.float32) + bl_ref[...]
    )
    xr_ref[...] = (
        jnp.dot(xb, wr_ref[...], preferred_element_type=jnp.float32) + br_ref[...]
    )


def _compute_chunk(heads, c_per_h, dim, wsc_s, webuf, attbuf,
                   xlr_s, xrr_s, msgb_s):
    ng = _CHUNK // _L
    rows_g = [g * _L + lax.iota(jnp.int32, _L) for g in range(ng)]
    w16_g = [plsc.load_gather(wsc_s, [rows_g[g]]) for g in range(ng)]

    def cbody(c, accs):
        out = list(accs)
        for h in range(heads):
            cc = h * c_per_h + c
            colv = jnp.full((_L,), cc, jnp.int32)
            wec = plsc.load_gather(webuf, [colv])
            atc = plsc.load_gather(attbuf, [colv])
            for g in range(ng):
                xlc = plsc.load_gather(xlr_s, [rows_g[g], colv])
                xrc = plsc.load_gather(xrr_s, [rows_g[g], colv])
                q = xlc + xrc + w16_g[g] * wec
                z = jnp.maximum(q, 0.2 * q)
                out[h * ng + g] = out[h * ng + g] + z * atc
        return tuple(out)

    accs = lax.fori_loop(
        0, c_per_h, cbody,
        tuple(jnp.zeros((_L,), jnp.float32) for _ in range(heads * ng)))
    exs = [jnp.exp(a) for a in accs]
    for h in range(heads):
        for g in range(ng):
            plsc.store_scatter(
                msgb_s, [rows_g[g], jnp.full((_L,), dim + h, jnp.int32)],
                exs[h * ng + g])

    def mbody(c, _):
        for h in range(heads):
            cc = h * c_per_h + c
            colv = jnp.full((_L,), cc, jnp.int32)
            for g in range(ng):
                xlc = plsc.load_gather(xlr_s, [rows_g[g], colv])
                plsc.store_scatter(msgb_s, [rows_g[g], colv],
                                   xlc * exs[h * ng + g])
        return 0

    lax.fori_loop(0, c_per_h, mbody, 0)


def _edge_body(n_chunks, dim, heads, c_per_h,
               xl_hbm, xr_hbm, src_hbm, dst_hbm, ew_hbm, we_hbm, att_hbm,
               zeros_hbm, out_hbm,
               sidx, didx, wslab, dsc, wsc, xlr, xrr, msgb, webuf, attbuf,
               accum,
               s_si0, s_si1, s_di0, s_di1, s_wi0, s_wi1,
               s_xl0, s_xl1, s_xr0, s_xr1, s_sc0, s_sc1):
    cid = lax.axis_index("c")
    sid = lax.axis_index("s")
    n_pad = out_hbm.shape[1]
    rows_per_tile = n_pad // _NS
    e_per_tile = n_chunks * _CHUNK

    s_si = (s_si0, s_si1)
    s_di = (s_di0, s_di1)
    s_wi = (s_wi0, s_wi1)
    s_xl = (s_xl0, s_xl1)
    s_xr = (s_xr0, s_xr1)
    s_sc = (s_sc0, s_sc1)

    base = (cid * _NS + sid) * e_per_tile
    pltpu.sync_copy(we_hbm, webuf)
    pltpu.sync_copy(att_hbm, attbuf)
    r0 = sid * rows_per_tile
    pltpu.sync_copy(zeros_hbm.at[pl.ds(r0, rows_per_tile)],
                    accum.at[pl.ds(r0, rows_per_tile)])
    plsc.subcore_barrier()

    def i_issue(j, s):
        e0 = base + j * _CHUNK
        pltpu.async_copy(src_hbm.at[pl.ds(e0, _CHUNK)], sidx.at[s], s_si[s])
        pltpu.async_copy(dst_hbm.at[pl.ds(e0, _CHUNK)], didx.at[s], s_di[s])
        pltpu.async_copy(ew_hbm.at[pl.ds(e0, _CHUNK)], wslab.at[s], s_wi[s])

    def i_wait(j, s):
        e0 = base + j * _CHUNK
        pltpu.make_async_copy(src_hbm.at[pl.ds(e0, _CHUNK)], sidx.at[s],
                              s_si[s]).wait()
        pltpu.make_async_copy(dst_hbm.at[pl.ds(e0, _CHUNK)], didx.at[s],
                              s_di[s]).wait()
        pltpu.make_async_copy(ew_hbm.at[pl.ds(e0, _CHUNK)], wslab.at[s],
                              s_wi[s]).wait()

    def g_issue(s):
        pltpu.async_copy(xl_hbm.at[sidx.at[s]], xlr.at[s], s_xl[s])
        pltpu.async_copy(xr_hbm.at[didx.at[s]], xrr.at[s], s_xr[s])

    def g_wait(s):
        pltpu.make_async_copy(xl_hbm.at[sidx.at[s]], xlr.at[s],
                              s_xl[s]).wait()
        pltpu.make_async_copy(xr_hbm.at[didx.at[s]], xrr.at[s],
                              s_xr[s]).wait()

    def save_idx(s):
        for g in range(_CHUNK // _L):
            rows = g * _L + lax.iota(jnp.int32, _L)
            dv = plsc.load_gather(didx.at[s], [rows])
            plsc.store_scatter(dsc.at[s], [rows], dv)
            wv = plsc.load_gather(wslab.at[s], [rows])
            plsc.store_scatter(wsc.at[s], [rows], wv)

    def sc_issue(s):
        pltpu.async_copy(msgb.at[s], accum.at[dsc.at[s]], s_sc[s], add=True)

    def sc_wait(s):
        pltpu.make_async_copy(msgb.at[s], accum.at[dsc.at[s]],
                              s_sc[s]).wait()

    def compute(s):
        _compute_chunk(heads, c_per_h, dim, wsc.at[s], webuf, attbuf,
                       xlr.at[s], xrr.at[s], msgb.at[s])

    i_issue(0, 0)
    i_issue(1, 1)
    i_wait(0, 0)
    g_issue(0)
    i_wait(1, 1)
    g_issue(1)
    g_wait(0)
    save_idx(0)
    i_issue(2, 0)
    compute(0)
    sc_issue(0)
    i_wait(2, 0)
    g_issue(0)
    g_wait(1)
    save_idx(1)
    i_issue(3, 1)
    compute(1)
    sc_issue(1)
    i_wait(3, 1)
    g_issue(1)

    n_pairs = (n_chunks - 5) // 2

    def step(j, s):
        g_wait(s)
        sc_wait(s)
        save_idx(s)
        i_issue(j + 2, s)
        compute(s)
        sc_issue(s)
        i_wait(j + 2, s)
        g_issue(s)

    def pair_body(k, carry):
        j = 2 + 2 * k
        step(j, 0)
        step(j + 1, 1)
        return carry

    lax.fori_loop(0, n_pairs, pair_body, 0)

    j = 2 + 2 * n_pairs
    step(j, 0)
    g_wait(1)
    sc_wait(1)
    save_idx(1)
    compute(1)
    sc_issue(1)
    g_wait(0)
    sc_wait(0)
    save_idx(0)
    compute(0)
    sc_issue(0)
    sc_wait(1)
    sc_wait(0)

    plsc.subcore_barrier()
    pltpu.sync_copy(accum.at[pl.ds(r0, rows_per_tile)],
                    out_hbm.at[cid, pl.ds(r0, rows_per_tile)])


def _fin_body(dim, heads, c_per_h, p0_ref, p1_ref, x_ref, b_ref, g_ref,
              bt_ref, o_ref):
    s = p0_ref[...] + p1_ref[...]
    pieces = []
    for h in range(heads):
        den = s[:, dim + h:dim + h + 1] + 1e-16
        pieces.append(s[:, h * c_per_h:(h + 1) * c_per_h] / den)
    out = jnp.concatenate(pieces, axis=1) + b_ref[...] + x_ref[...]
    mu = jnp.mean(out, axis=1, keepdims=True)
    var = jnp.mean((out - mu) * (out - mu), axis=1, keepdims=True)
    o_ref[...] = (out - mu) * lax.rsqrt(var + 1e-5) * g_ref[...] + bt_ref[...]


def kernel(x, edge_index, edge_weight, W_l, b_l, W_r, b_r, W_e, att, bias,
           gamma, beta):
    n, dim = x.shape
    e = edge_index.shape[1]
    heads, c_per_h = att.shape
    dn = dim + heads

    src = edge_index[0].astype(jnp.int32)
    dst = edge_index[1].astype(jnp.int32)
    ew = edge_weight.reshape(e).astype(jnp.float32)
    we = W_e.reshape(dim)
    attb = att.reshape(dim)

    rb = 400
    grid = (n // rb,)
    xl, xr = pl.pallas_call(
        _proj_body,
        grid=grid,
        in_specs=[
            pl.BlockSpec((rb, dim), lambda i: (i, 0)),
            pl.BlockSpec((dim, dim), lambda i: (0, 0)),
            pl.BlockSpec((dim, dim), lambda i: (0, 0)),
            pl.BlockSpec((1, dim), lambda i: (0, 0)),
            pl.BlockSpec((1, dim), lambda i: (0, 0)),
        ],
        out_specs=[
            pl.BlockSpec((rb, dim), lambda i: (i, 0)),
            pl.BlockSpec((rb, dim), lambda i: (i, 0)),
        ],
        out_shape=[
            jax.ShapeDtypeStruct((n, dim), jnp.float32),
            jax.ShapeDtypeStruct((n, dim), jnp.float32),
        ],
    )(x, W_l, W_r, b_l.reshape(1, dim), b_r.reshape(1, dim))

    n_tiles = _NC * _NS
    n_chunks = -(-e // (n_tiles * _CHUNK))
    if (n_chunks - 5) % 2:
        n_chunks += 1
    e_pad = n_tiles * _CHUNK * n_chunks
    pad = e_pad - e
    srcp = jnp.concatenate([src, jnp.zeros((pad,), jnp.int32)])
    dstp = jnp.concatenate([dst, jnp.full((pad,), n, jnp.int32)])
    ewp = jnp.concatenate([ew, jnp.zeros((pad,), jnp.float32)])
    n_pad = n + _NS
    xlp = jnp.pad(xl, ((0, n_pad - n), (0, 0)))
    xrp = jnp.pad(xr, ((0, n_pad - n), (0, 0)))
    zeros = jnp.zeros((n_pad, dn), jnp.float32)

    mesh = plsc.VectorSubcoreMesh(core_axis_name="c", subcore_axis_name="s")
    parts = pl.kernel(
        functools.partial(_edge_body, n_chunks, dim, heads, c_per_h),
        out_type=jax.ShapeDtypeStruct((_NC, n_pad, dn), jnp.float32),
        mesh=mesh,
        compiler_params=pltpu.CompilerParams(
            use_tc_tiling_on_sc=False, needs_layout_passes=False),
        scratch_types=[
            pltpu.VMEM((2, _CHUNK), jnp.int32),
            pltpu.VMEM((2, _CHUNK), jnp.int32),
            pltpu.VMEM((2, _CHUNK), jnp.float32),
            pltpu.VMEM((2, _CHUNK), jnp.int32),
            pltpu.VMEM((2, _CHUNK), jnp.float32),
            pltpu.VMEM((2, _CHUNK, dim), jnp.float32),
            pltpu.VMEM((2, _CHUNK, dim), jnp.float32),
            pltpu.VMEM((2, _CHUNK, dn), jnp.float32),
            pltpu.VMEM((dim,), jnp.float32),
            pltpu.VMEM((dim,), jnp.float32),
            pltpu.VMEM_SHARED((n_pad, dn), jnp.float32),
            pltpu.SemaphoreType.DMA,
            pltpu.SemaphoreType.DMA,
            pltpu.SemaphoreType.DMA,
            pltpu.SemaphoreType.DMA,
            pltpu.SemaphoreType.DMA,
            pltpu.SemaphoreType.DMA,
            pltpu.SemaphoreType.DMA,
            pltpu.SemaphoreType.DMA,
            pltpu.SemaphoreType.DMA,
            pltpu.SemaphoreType.DMA,
            pltpu.SemaphoreType.DMA,
            pltpu.SemaphoreType.DMA,
        ],
    )(xlp, xrp, srcp, dstp, ewp, we, attb, zeros)

    out = pl.pallas_call(
        functools.partial(_fin_body, dim, heads, c_per_h),
        grid=grid,
        in_specs=[
            pl.BlockSpec((rb, dn), lambda i: (i, 0)),
            pl.BlockSpec((rb, dn), lambda i: (i, 0)),
            pl.BlockSpec((rb, dim), lambda i: (i, 0)),
            pl.BlockSpec((1, dim), lambda i: (0, 0)),
            pl.BlockSpec((1, dim), lambda i: (0, 0)),
            pl.BlockSpec((1, dim), lambda i: (0, 0)),
        ],
        out_specs=pl.BlockSpec((rb, dim), lambda i: (i, 0)),
        out_shape=jax.ShapeDtypeStruct((n, dim), jnp.float32),
    )(parts[0], parts[1], x, bias.reshape(1, dim), gamma.reshape(1, dim),
      beta.reshape(1, dim))
    return out


# speedup vs baseline: 1.0251x; 1.0251x over previous
import functools

import jax
import jax.numpy as jnp
from jax import lax
from jax.experimental import pallas as pl
from jax.experimental.pallas import tpu as pltpu
from jax.experimental.pallas import tpu_sc as plsc

_NC = 2
_NS = 16
_L = 16
_CHUNK = 48


def _proj_body(x_ref, wl_ref, wr_ref, bl_ref, br_ref, xl_ref, xr_ref):
    xb = x_ref[...]
    xl_ref[...] = (
        jnp.dot(xb, wl_ref[...], preferred_element_type=jnp.float32) + bl_ref[...]
    )
    xr_ref[...] = (
        jnp.dot(xb, wr_ref[...], preferred_element_type=jnp.float32) + br_ref[...]
    )


def _compute_chunk(heads, c_per_h, dim, wsc_s, webuf, attbuf,
                   xlr_s, xrr_s, msgb_s):
    def group_body(g, _):
        rows = g * _L + lax.iota(jnp.int32, _L)
        w16 = plsc.load_gather(wsc_s, [rows])

        def cbody(c, accs):
            out = []
            for h in range(heads):
                cc = h * c_per_h + c
                colv = jnp.full((_L,), cc, jnp.int32)
                xlc = plsc.load_gather(xlr_s, [rows, colv])
                xrc = plsc.load_gather(xrr_s, [rows, colv])
                wec = plsc.load_gather(webuf, [colv])
                atc = plsc.load_gather(attbuf, [colv])
                q = xlc + xrc + w16 * wec
                z = jnp.maximum(q, 0.2 * q)
                out.append(accs[h] + z * atc)
            return tuple(out)

        accs = lax.fori_loop(
            0, c_per_h, cbody,
            tuple(jnp.zeros((_L,), jnp.float32) for _ in range(heads)))
        exs = [jnp.exp(a) for a in accs]
        for h in range(heads):
            plsc.store_scatter(
                msgb_s, [rows, jnp.full((_L,), dim + h, jnp.int32)], exs[h])

        def mbody(c, _):
            for h in range(heads):
                cc = h * c_per_h + c
                colv = jnp.full((_L,), cc, jnp.int32)
                xlc = plsc.load_gather(xlr_s, [rows, colv])
                plsc.store_scatter(msgb_s, [rows, colv], xlc * exs[h])
            return 0

        lax.fori_loop(0, c_per_h, mbody, 0)
        return 0

    lax.fori_loop(0, _CHUNK // _L, group_body, 0)


def _edge_body(n_chunks, dim, heads, c_per_h,
               xl_hbm, xr_hbm, src_hbm, dst_hbm, ew_hbm, we_hbm, att_hbm,
               zeros_hbm, out_hbm,
               sidx, didx, wslab, dsc, wsc, xlr, xrr, msgb, webuf, attbuf,
               accum,
               s_si0, s_si1, s_di0, s_di1, s_wi0, s_wi1,
               s_xl0, s_xl1, s_xr0, s_xr1, s_sc0, s_sc1):
    cid = lax.axis_index("c")
    sid = lax.axis_index("s")
    n_pad = out_hbm.shape[1]
    rows_per_tile = n_pad // _NS
    e_per_tile = n_chunks * _CHUNK

    s_si = (s_si0, s_si1)
    s_di = (s_di0, s_di1)
    s_wi = (s_wi0, s_wi1)
    s_xl = (s_xl0, s_xl1)
    s_xr = (s_xr0, s_xr1)
    s_sc = (s_sc0, s_sc1)

    base = (cid * _NS + sid) * e_per_tile
    pltpu.sync_copy(we_hbm, webuf)
    pltpu.sync_copy(att_hbm, attbuf)
    r0 = sid * rows_per_tile
    pltpu.sync_copy(zeros_hbm.at[pl.ds(r0, rows_per_tile)],
                    accum.at[pl.ds(r0, rows_per_tile)])
    plsc.subcore_barrier()

    def i_issue(j, s):
        e0 = base + j * _CHUNK
        pltpu.async_copy(src_hbm.at[pl.ds(e0, _CHUNK)], sidx.at[s], s_si[s])
        pltpu.async_copy(dst_hbm.at[pl.ds(e0, _CHUNK)], didx.at[s], s_di[s])
        pltpu.async_copy(ew_hbm.at[pl.ds(e0, _CHUNK)], wslab.at[s], s_wi[s])

    def i_wait(j, s):
        e0 = base + j * _CHUNK
        pltpu.make_async_copy(src_hbm.at[pl.ds(e0, _CHUNK)], sidx.at[s],
                              s_si[s]).wait()
        pltpu.make_async_copy(dst_hbm.at[pl.ds(e0, _CHUNK)], didx.at[s],
                              s_di[s]).wait()
        pltpu.make_async_copy(ew_hbm.at[pl.ds(e0, _CHUNK)], wslab.at[s],
                              s_wi[s]).wait()

    def g_issue(s):
        pltpu.async_copy(xl_hbm.at[sidx.at[s]], xlr.at[s], s_xl[s])
        pltpu.async_copy(xr_hbm.at[didx.at[s]], xrr.at[s], s_xr[s])

    def g_wait(s):
        pltpu.make_async_copy(xl_hbm.at[sidx.at[s]], xlr.at[s],
                              s_xl[s]).wait()
        pltpu.make_async_copy(xr_hbm.at[didx.at[s]], xrr.at[s],
                              s_xr[s]).wait()

    def save_idx(s):
        for g in range(_CHUNK // _L):
            rows = g * _L + lax.iota(jnp.int32, _L)
            dv = plsc.load_gather(didx.at[s], [rows])
            plsc.store_scatter(dsc.at[s], [rows], dv)
            wv = plsc.load_gather(wslab.at[s], [rows])
            plsc.store_scatter(wsc.at[s], [rows], wv)

    def sc_issue(s):
        pltpu.async_copy(msgb.at[s], accum.at[dsc.at[s]], s_sc[s], add=True)

    def sc_wait(s):
        pltpu.make_async_copy(msgb.at[s], accum.at[dsc.at[s]],
                              s_sc[s]).wait()

    def compute(s):
        _compute_chunk(heads, c_per_h, dim, wsc.at[s], webuf, attbuf,
                       xlr.at[s], xrr.at[s], msgb.at[s])

    i_issue(0, 0)
    i_issue(1, 1)
    i_wait(0, 0)
    g_issue(0)
    i_wait(1, 1)
    g_issue(1)
    g_wait(0)
    save_idx(0)
    i_issue(2, 0)
    compute(0)
    sc_issue(0)
    i_wait(2, 0)
    g_issue(0)
    g_wait(1)
    save_idx(1)
    i_issue(3, 1)
    compute(1)
    sc_issue(1)
    i_wait(3, 1)
    g_issue(1)

    n_pairs = (n_chunks - 5) // 2

    def step(j, s):
        g_wait(s)
        sc_wait(s)
        save_idx(s)
        i_issue(j + 2, s)
        compute(s)
        sc_issue(s)
        i_wait(j + 2, s)
        g_issue(s)

    def pair_body(k, carry):
        j = 2 + 2 * k
        step(j, 0)
        step(j + 1, 1)
        return carry

    lax.fori_loop(0, n_pairs, pair_body, 0)

    j = 2 + 2 * n_pairs
    step(j, 0)
    g_wait(1)
    sc_wait(1)
    save_idx(1)
    compute(1)
    sc_issue(1)
    g_wait(0)
    sc_wait(0)
    save_idx(0)
    compute(0)
    sc_issue(0)
    sc_wait(1)
    sc_wait(0)

    plsc.subcore_barrier()
    pltpu.sync_copy(accum.at[pl.ds(r0, rows_per_tile)],
                    out_hbm.at[cid, pl.ds(r0, rows_per_tile)])


def _fin_body(dim, heads, c_per_h, p0_ref, p1_ref, x_ref, b_ref, g_ref,
              bt_ref, o_ref):
    s = p0_ref[...] + p1_ref[...]
    pieces = []
    for h in range(heads):
        den = s[:, dim + h:dim + h + 1] + 1e-16
        pieces.append(s[:, h * c_per_h:(h + 1) * c_per_h] / den)
    out = jnp.concatenate(pieces, axis=1) + b_ref[...] + x_ref[...]
    mu = jnp.mean(out, axis=1, keepdims=True)
    var = jnp.mean((out - mu) * (out - mu), axis=1, keepdims=True)
    o_ref[...] = (out - mu) * lax.rsqrt(var + 1e-5) * g_ref[...] + bt_ref[...]


def kernel(x, edge_index, edge_weight, W_l, b_l, W_r, b_r, W_e, att, bias,
           gamma, beta):
    n, dim = x.shape
    e = edge_index.shape[1]
    heads, c_per_h = att.shape
    dn = dim + heads

    src = edge_index[0].astype(jnp.int32)
    dst = edge_index[1].astype(jnp.int32)
    ew = edge_weight.reshape(e).astype(jnp.float32)
    we = W_e.reshape(dim)
    attb = att.reshape(dim)

    rb = 400
    grid = (n // rb,)
    xl, xr = pl.pallas_call(
        _proj_body,
        grid=grid,
        in_specs=[
            pl.BlockSpec((rb, dim), lambda i: (i, 0)),
            pl.BlockSpec((dim, dim), lambda i: (0, 0)),
            pl.BlockSpec((dim, dim), lambda i: (0, 0)),
            pl.BlockSpec((1, dim), lambda i: (0, 0)),
            pl.BlockSpec((1, dim), lambda i: (0, 0)),
        ],
        out_specs=[
            pl.BlockSpec((rb, dim), lambda i: (i, 0)),
            pl.BlockSpec((rb, dim), lambda i: (i, 0)),
        ],
        out_shape=[
            jax.ShapeDtypeStruct((n, dim), jnp.float32),
            jax.ShapeDtypeStruct((n, dim), jnp.float32),
        ],
    )(x, W_l, W_r, b_l.reshape(1, dim), b_r.reshape(1, dim))

    n_tiles = _NC * _NS
    n_chunks = -(-e // (n_tiles * _CHUNK))
    if (n_chunks - 5) % 2:
        n_chunks += 1
    e_pad = n_tiles * _CHUNK * n_chunks
    pad = e_pad - e
    srcp = jnp.concatenate([src, jnp.zeros((pad,), jnp.int32)])
    dstp = jnp.concatenate([dst, jnp.full((pad,), n, jnp.int32)])
    ewp = jnp.concatenate([ew, jnp.zeros((pad,), jnp.float32)])
    n_pad = n + _NS
    xlp = jnp.pad(xl, ((0, n_pad - n), (0, 0)))
    xrp = jnp.pad(xr, ((0, n_pad - n), (0, 0)))
    zeros = jnp.zeros((n_pad, dn), jnp.float32)

    mesh = plsc.VectorSubcoreMesh(core_axis_name="c", subcore_axis_name="s")
    parts = pl.kernel(
        functools.partial(_edge_body, n_chunks, dim, heads, c_per_h),
        out_type=jax.ShapeDtypeStruct((_NC, n_pad, dn), jnp.float32),
        mesh=mesh,
        compiler_params=pltpu.CompilerParams(
            use_tc_tiling_on_sc=False, needs_layout_passes=False),
        scratch_types=[
            pltpu.VMEM((2, _CHUNK), jnp.int32),
            pltpu.VMEM((2, _CHUNK), jnp.int32),
            pltpu.VMEM((2, _CHUNK), jnp.float32),
            pltpu.VMEM((2, _CHUNK), jnp.int32),
            pltpu.VMEM((2, _CHUNK), jnp.float32),
            pltpu.VMEM((2, _CHUNK, dim), jnp.float32),
            pltpu.VMEM((2, _CHUNK, dim), jnp.float32),
            pltpu.VMEM((2, _CHUNK, dn), jnp.float32),
            pltpu.VMEM((dim,), jnp.float32),
            pltpu.VMEM((dim,), jnp.float32),
            pltpu.VMEM_SHARED((n_pad, dn), jnp.float32),
            pltpu.SemaphoreType.DMA,
            pltpu.SemaphoreType.DMA,
            pltpu.SemaphoreType.DMA,
            pltpu.SemaphoreType.DMA,
            pltpu.SemaphoreType.DMA,
            pltpu.SemaphoreType.DMA,
            pltpu.SemaphoreType.DMA,
            pltpu.SemaphoreType.DMA,
            pltpu.SemaphoreType.DMA,
            pltpu.SemaphoreType.DMA,
            pltpu.SemaphoreType.DMA,
            pltpu.SemaphoreType.DMA,
        ],
    )(xlp, xrp, srcp, dstp, ewp, we, attb, zeros)

    out = pl.pallas_call(
        functools.partial(_fin_body, dim, heads, c_per_h),
        grid=grid,
        in_specs=[
            pl.BlockSpec((rb, dn), lambda i: (i, 0)),
            pl.BlockSpec((rb, dn), lambda i: (i, 0)),
            pl.BlockSpec((rb, dim), lambda i: (i, 0)),
            pl.BlockSpec((1, dim), lambda i: (0, 0)),
            pl.BlockSpec((1, dim), lambda i: (0, 0)),
            pl.BlockSpec((1, dim), lambda i: (0, 0)),
        ],
        out_specs=pl.BlockSpec((rb, dim), lambda i: (i, 0)),
        out_shape=jax.ShapeDtypeStruct((n, dim), jnp.float32),
    )(parts[0], parts[1], x, bias.reshape(1, dim), gamma.reshape(1, dim),
      beta.reshape(1, dim))
    return out
